# Initial kernel scaffold; baseline (speedup 1.0000x reference)
import jax
import jax.numpy as jnp
from jax.experimental import pallas as pl


def kernel(feature, edge_index, edge_weight, W, b, gamma, beta):
    raise NotImplementedError("write your pallas kernel here")


# speedup vs baseline: 4.3450x; 4.3450x over previous
import functools

import jax
import jax.numpy as jnp
from jax import lax
from jax.experimental import pallas as pl
from jax.experimental.pallas import tpu as pltpu
from jax.experimental.pallas import tpu_sc as plsc

_N = 10000
_E = 320000
_D = 128

_NW = 32
_C = 128
_KPW = 79
_EPAD = _NW * _KPW * _C
_RPW = _N // 16


def _aggregate_sc(feature, src2d, dst2d, w2d):
    mesh = plsc.VectorSubcoreMesh(core_axis_name="c", subcore_axis_name="s")

    @functools.partial(
        pl.kernel,
        mesh=mesh,
        out_type=jax.ShapeDtypeStruct((2, _N, _D), jnp.float32),
        scratch_types=[
            pltpu.VMEM((_KPW, _C), jnp.int32),
            pltpu.VMEM((_KPW, _C), jnp.int32),
            pltpu.VMEM((_KPW * _C,), jnp.float32),
            pltpu.VMEM((_C, _D), jnp.float32),
            pltpu.VMEM_SHARED((_N, _D), jnp.float32),
            pltpu.SemaphoreType.DMA,
        ],
    )
    def body(feat_hbm, src_hbm, dst_hbm, w_hbm, out_hbm,
             src_v, dst_v, w_v, rows_v, acc_sh, sem):
        c = lax.axis_index("c")
        s = lax.axis_index("s")
        wkr = s * 2 + c

        z16 = jnp.zeros((16,), jnp.float32)

        def _zrow(r, carry):
            for j in range(_D // 16):
                rows_v[r, pl.ds(j * 16, 16)] = z16
            return carry

        lax.fori_loop(0, _C, _zrow, 0)
        rows_per_copy = 125
        for j in range(_RPW // rows_per_copy):
            pltpu.sync_copy(
                rows_v.at[pl.ds(0, rows_per_copy)],
                acc_sh.at[pl.ds(s * _RPW + j * rows_per_copy, rows_per_copy)],
            )
        plsc.subcore_barrier()

        pltpu.sync_copy(src_hbm.at[wkr], src_v)
        pltpu.sync_copy(dst_hbm.at[wkr], dst_v)
        pltpu.sync_copy(w_hbm.at[pl.ds(wkr * _KPW * _C, _KPW * _C)], w_v)

        def _chunk(k, carry):
            pltpu.async_copy(feat_hbm.at[src_v.at[k]], rows_v, sem).wait()

            def _grp(g, carry2):
                w16 = w_v[pl.ds(k * _C + g * 16, 16)]

                def _lane(i, carry3):
                    wspl = lax.gather(
                        w16, jnp.full((16, 1), i, jnp.int32),
                        lax.GatherDimensionNumbers(
                            offset_dims=(), collapsed_slice_dims=(0,),
                            start_index_map=(0,)),
                        (1,), mode=lax.GatherScatterMode.PROMISE_IN_BOUNDS)
                    e = g * 16 + i
                    for j in range(_D // 16):
                        sl = pl.ds(j * 16, 16)
                        rows_v[e, sl] = rows_v[e, sl] * wspl
                    return carry3

                lax.fori_loop(0, 16, _lane, 0)
                return carry2

            lax.fori_loop(0, _C // 16, _grp, 0)

            pltpu.sync_copy(rows_v, acc_sh.at[dst_v.at[k]], add=True)
            return carry

        lax.fori_loop(0, _KPW, _chunk, 0)
        plsc.subcore_barrier()

        nchunks = _N // 80
        for j in range(8):
            k = s + 16 * j

            @pl.when(k < nchunks)
            def _():
                r = k * 80
                pltpu.sync_copy(acc_sh.at[pl.ds(r, 80)],
                                rows_v.at[pl.ds(0, 80)])
                pltpu.sync_copy(rows_v.at[pl.ds(0, 80)],
                                out_hbm.at[c, pl.ds(r, 80)])

    return body(feature, src2d, dst2d, w2d)


def _dense_body(p0_ref, p1_ref, w_ref, b_ref, g_ref, bt_ref, o_ref):
    h = p0_ref[...] + p1_ref[...]
    y = lax.dot_general(h, w_ref[...], (((1,), (1,)), ((), ())),
                        preferred_element_type=jnp.float32)
    y = jnp.maximum(y + b_ref[...], 0.0)
    mean = jnp.mean(y, axis=0, keepdims=True)
    var = jnp.mean(jnp.square(y - mean), axis=0, keepdims=True)
    o_ref[...] = (y - mean) / jnp.sqrt(var + 1e-5) * g_ref[...] + bt_ref[...]


def kernel(feature, edge_index, edge_weight, W, b, gamma, beta):
    src = edge_index[0].astype(jnp.int32)
    dst = edge_index[1].astype(jnp.int32)
    w = edge_weight.reshape(_E).astype(jnp.float32)
    pad = _EPAD - _E
    src2d = jnp.concatenate([src, jnp.zeros((pad,), jnp.int32)]).reshape(_NW, _KPW, _C)
    dst2d = jnp.concatenate([dst, jnp.zeros((pad,), jnp.int32)]).reshape(_NW, _KPW, _C)
    w2d = jnp.concatenate([w, jnp.zeros((pad,), jnp.float32)])

    partials = _aggregate_sc(feature, src2d, dst2d, w2d)

    out = pl.pallas_call(
        _dense_body,
        out_shape=jax.ShapeDtypeStruct((_N, _D), jnp.float32),
    )(partials[0], partials[1], W,
      b.reshape(1, _D), gamma.reshape(1, _D), beta.reshape(1, _D))
    return out
